# baseline (speedup 1.0000x reference)
import functools

import jax
import jax.numpy as jnp
from jax import lax
from jax.experimental import pallas as pl
from jax.experimental.pallas import tpu as pltpu
from jax.experimental.pallas import tpu_sc as plsc

T_DIM, H_DIM, W_DIM, EMBED_DIM = 16, 24, 24, 384
BATCH = 8
L = 16
NWORK = 32
ROWS = BATCH * T_DIM * H_DIM * W_DIM
GROUPS = ROWS // W_DIM
GPW = GROUPS // NWORK
GPC = 2
CHUNKS = GPW // GPC
CROWS = GPC * W_DIM
NBUF = 4
NCOL = EMBED_DIM // L


def _in_start(x_hbm, buf, sems, slot, k, base_row):
    pltpu.async_copy(
        x_hbm.at[pl.ds(base_row + k * CROWS, CROWS), :], buf.at[slot], sems[slot]
    )


def _in_wait(x_hbm, buf, sems, slot, base_row):
    pltpu.make_async_copy(
        x_hbm.at[pl.ds(base_row, CROWS), :], buf.at[slot], sems[slot]
    ).wait()


def _out_start(out_hbm, buf, sems, slot, k, base_row):
    pltpu.async_copy(
        buf.at[slot], out_hbm.at[pl.ds(base_row + k * CROWS, CROWS), :],
        sems[NBUF + slot],
    )


def _out_wait(out_hbm, buf, sems, slot, base_row):
    pltpu.make_async_copy(
        buf.at[slot], out_hbm.at[pl.ds(base_row, CROWS), :], sems[NBUF + slot]
    ).wait()


def _compute(buf, tv, hv, wv, slot, k, base_group):
    gid0 = base_group + k * GPC
    ts, hs = [], []
    for g in range(GPC):
        rem = lax.rem(gid0 + g, T_DIM * H_DIM)
        ts.append(lax.div(rem, H_DIM))
        hs.append(lax.rem(rem, H_DIM))

    def col_body(c, carry):
        off = pl.ds(c * L, L)
        wcol = [wv[w, off] for w in range(W_DIM)]
        for g in range(GPC):
            th = tv[ts[g], off] * hv[hs[g], off]
            for w in range(W_DIM):
                r = g * W_DIM + w
                buf[slot, r, off] = buf[slot, r, off] + th * wcol[w]
        return carry

    lax.fori_loop(0, NCOL, col_body, 0)


def kernel(x, t_embed, h_embed, w_embed):
    xr = x.reshape(ROWS, EMBED_DIM)
    mesh = plsc.VectorSubcoreMesh(core_axis_name="c", subcore_axis_name="s")

    @functools.partial(
        pl.kernel,
        mesh=mesh,
        out_type=jax.ShapeDtypeStruct((ROWS, EMBED_DIM), jnp.float32),
        scratch_types=[
            pltpu.VMEM((NBUF, CROWS, EMBED_DIM), jnp.float32),
            pltpu.VMEM((T_DIM, EMBED_DIM), jnp.float32),
            pltpu.VMEM((H_DIM, EMBED_DIM), jnp.float32),
            pltpu.VMEM((W_DIM, EMBED_DIM), jnp.float32),
        ] + [pltpu.SemaphoreType.DMA] * (2 * NBUF),
    )
    def sc_add(x_hbm, t_hbm, h_hbm, w_hbm, out_hbm, buf, tv, hv, wv, *sems):
        wid = lax.axis_index("s") * 2 + lax.axis_index("c")
        base_group = wid * GPW
        base_row = base_group * W_DIM
        pltpu.sync_copy(t_hbm, tv)
        pltpu.sync_copy(h_hbm, hv)
        pltpu.sync_copy(w_hbm, wv)

        _in_start(x_hbm, buf, sems, 0, 0, base_row)
        _in_start(x_hbm, buf, sems, 1, 1, base_row)

        for k in (0, 1):
            _in_start(x_hbm, buf, sems, k + 2, k + 2, base_row)
            _in_wait(x_hbm, buf, sems, k, base_row)
            _compute(buf, tv, hv, wv, k, k, base_group)
            _out_start(out_hbm, buf, sems, k, k, base_row)

        def mid(m, carry):
            k0 = 2 + m * NBUF
            for j in range(NBUF):
                k = k0 + j
                slot = (2 + j) % NBUF
                nslot = j % NBUF
                _out_wait(out_hbm, buf, sems, nslot, base_row)
                _in_start(x_hbm, buf, sems, nslot, k + 2, base_row)
                _in_wait(x_hbm, buf, sems, slot, base_row)
                _compute(buf, tv, hv, wv, slot, k, base_group)
                _out_start(out_hbm, buf, sems, slot, k, base_row)
            return carry

        lax.fori_loop(0, (CHUNKS - 4) // NBUF, mid, 0)

        for k in (CHUNKS - 2, CHUNKS - 1):
            slot = k % NBUF
            _out_wait(out_hbm, buf, sems, (k + 2) % NBUF, base_row)
            _in_wait(x_hbm, buf, sems, slot, base_row)
            _compute(buf, tv, hv, wv, slot, k, base_group)
            _out_start(out_hbm, buf, sems, slot, k, base_row)

        for k in (CHUNKS - 2, CHUNKS - 1):
            _out_wait(out_hbm, buf, sems, k % NBUF, base_row)

    out = sc_add(xr, t_embed, h_embed, w_embed)
    return out.reshape(x.shape)
